# baseline (device time: 91551 ns/iter reference)
import jax
import jax.numpy as jnp
from jax import lax
from jax.experimental import pallas as pl
from jax.experimental.pallas import tpu as pltpu

T_LOC = 512
D = 1024
F = 2048
E_LOC = 4
E = 8


def _top2_weights(g):
    m1 = jnp.max(g, axis=1, keepdims=True)
    is1 = g == m1
    gneg = jnp.where(is1, -jnp.inf, g)
    m2 = jnp.max(gneg, axis=1, keepdims=True)
    is2 = gneg == m2
    e2 = jnp.exp(m2 - m1)
    w1 = 1.0 / (1.0 + e2)
    w2 = e2 / (1.0 + e2)
    return is1 * w1 + is2 * w2


def kernel(x, router, W1, W2):
    def body(x_ref, r_ref, w1_ref, w2_ref, out_ref,
             xs_bf, xp_bf, rrecv, gsend, grecv, psend, precv,
             accp, wo_ref, wp_ref, sems):
        e = pl.program_id(0)
        my_x = lax.axis_index("x")
        my_y = lax.axis_index("y")
        my_z = lax.axis_index("z")
        peer = (1 - my_x, my_y, my_z)

        def rd(src, dst, s0, s1):
            return pltpu.make_async_remote_copy(
                src_ref=src, dst_ref=dst,
                send_sem=sems.at[s0], recv_sem=sems.at[s1],
                device_id=peer, device_id_type=pl.DeviceIdType.MESH)

        rd_x = rd(xs_bf, xp_bf, 0, 1)
        rd_r = rd(r_ref, rrecv, 2, 3)
        rd_g = rd(gsend, grecv, 4, 5)
        rd_p = rd(psend, precv, 6, 7)

        @pl.when(e == 0)
        def _prologue():
            barrier = pltpu.get_barrier_semaphore()
            pl.semaphore_signal(barrier, inc=1, device_id=peer,
                                device_id_type=pl.DeviceIdType.MESH)
            pl.semaphore_wait(barrier, 1)

            xs_bf[...] = x_ref[...].astype(jnp.bfloat16)
            rd_x.start()
            rd_r.start()
            rd_r.wait()

            xf = x_ref[...]
            dot_f32 = lambda a, b: lax.dot_general(
                a, b, (((1,), (0,)), ((), ())),
                precision=lax.Precision.HIGHEST)
            g_loc = dot_f32(xf, r_ref[...])
            g_pe = dot_f32(xf, rrecv[...])
            g_own = jnp.where(my_x == 0,
                              jnp.concatenate([g_loc, g_pe], axis=1),
                              jnp.concatenate([g_pe, g_loc], axis=1))
            gsend[...] = g_own
            rd_g.start()
            wd_own = _top2_weights(g_own)
            wo_ref[...] = jnp.where(my_x == 0, wd_own[:, :E_LOC],
                                    wd_own[:, E_LOC:])

        w1e = w1_ref[0].astype(jnp.bfloat16)
        w2e = w2_ref[0].astype(jnp.bfloat16)

        def ffn(xblk):
            o = jnp.zeros((T_LOC, D), jnp.float32)
            for f0 in (0, F // 2):
                h = jnp.dot(xblk, w1e[:, f0:f0 + F // 2],
                            preferred_element_type=jnp.float32)
                h = jnp.maximum(h, 0.0).astype(jnp.bfloat16)
                o = o + jnp.dot(h, w2e[f0:f0 + F // 2, :],
                                preferred_element_type=jnp.float32)
            return o

        @pl.when(e == 0)
        def _step0():
            out_ref[...] = ffn(xs_bf[...]) * wo_ref[:, 0][:, None]
            rd_x.wait()
            rd_g.wait()
            wd_peer = _top2_weights(grecv[...])
            wp_ref[...] = jnp.where(my_x == 0, wd_peer[:, :E_LOC],
                                    wd_peer[:, E_LOC:])
            accp[...] = ffn(xp_bf[...]) * wp_ref[:, 0][:, None]

        @pl.when((e == 1) | (e == 2))
        def _mid():
            onehot = (lax.broadcasted_iota(jnp.int32, (1, E_LOC), 1) == e)
            woc = jnp.sum(wo_ref[...] * onehot, axis=1, keepdims=True)
            wpc = jnp.sum(wp_ref[...] * onehot, axis=1, keepdims=True)
            out_ref[...] = out_ref[...] + ffn(xs_bf[...]) * woc
            accp[...] = accp[...] + ffn(xp_bf[...]) * wpc

        @pl.when(e == E_LOC - 1)
        def _last():
            pa = accp[...] + ffn(xp_bf[...]) * wp_ref[:, E_LOC - 1][:, None]
            psend[...] = pa.astype(jnp.bfloat16)
            rd_p.start()
            own = ffn(xs_bf[...]) * wo_ref[:, E_LOC - 1][:, None]
            rd_p.wait()
            out_ref[...] = out_ref[...] + own \
                + precv[...].astype(jnp.float32)

    return pl.pallas_call(
        body,
        grid=(E_LOC,),
        out_shape=jax.ShapeDtypeStruct((T_LOC, D), jnp.float32),
        in_specs=[
            pl.BlockSpec((T_LOC, D), lambda e: (0, 0)),
            pl.BlockSpec((D, E_LOC), lambda e: (0, 0)),
            pl.BlockSpec((1, D, F), lambda e: (e, 0, 0)),
            pl.BlockSpec((1, F, D), lambda e: (e, 0, 0)),
        ],
        out_specs=pl.BlockSpec((T_LOC, D), lambda e: (0, 0)),
        scratch_shapes=[
            pltpu.VMEM((T_LOC, D), jnp.bfloat16),
            pltpu.VMEM((T_LOC, D), jnp.bfloat16),
            pltpu.VMEM((D, E_LOC), jnp.float32),
            pltpu.VMEM((T_LOC, E), jnp.float32),
            pltpu.VMEM((T_LOC, E), jnp.float32),
            pltpu.VMEM((T_LOC, D), jnp.bfloat16),
            pltpu.VMEM((T_LOC, D), jnp.bfloat16),
            pltpu.VMEM((T_LOC, D), jnp.float32),
            pltpu.VMEM((T_LOC, E_LOC), jnp.float32),
            pltpu.VMEM((T_LOC, E_LOC), jnp.float32),
            pltpu.SemaphoreType.DMA((8,)),
        ],
        compiler_params=pltpu.CompilerParams(
            collective_id=0, vmem_limit_bytes=60 * 1024 * 1024),
    )(x, router, W1, W2)
